# baseline (speedup 1.0000x reference)
import functools

import jax
import jax.numpy as jnp
from jax.experimental import pallas as pl

N_NODES = 10000
N_EDGES = 160000
HID = 512
M_TILE = 1000


def _mm_body(mean_ref, h_ref, wl_ref, bl_ref, wr_ref, invc_ref, out_ref):
    mean = mean_ref[...] * invc_ref[...]
    acc = jnp.dot(mean, wl_ref[...], preferred_element_type=jnp.float32)
    acc += jnp.dot(h_ref[...], wr_ref[...], preferred_element_type=jnp.float32)
    out_ref[...] = acc + bl_ref[...]


@functools.partial(jax.jit, static_argnames=("c_in",))
def _sage_mm(agg, h, wl, bl, wr, invc, c_in):
    grid = (N_NODES // M_TILE,)
    return pl.pallas_call(
        _mm_body,
        grid=grid,
        in_specs=[
            pl.BlockSpec((M_TILE, c_in), lambda i: (i, 0)),
            pl.BlockSpec((M_TILE, c_in), lambda i: (i, 0)),
            pl.BlockSpec((c_in, HID), lambda i: (0, 0)),
            pl.BlockSpec((1, HID), lambda i: (0, 0)),
            pl.BlockSpec((c_in, HID), lambda i: (0, 0)),
            pl.BlockSpec((M_TILE, 1), lambda i: (i, 0)),
        ],
        out_specs=pl.BlockSpec((M_TILE, HID), lambda i: (i, 0)),
        out_shape=jax.ShapeDtypeStruct((N_NODES, HID), jnp.float32),
    )(agg, h, wl, bl, wr, invc)


def kernel(x, adj_t, Wl0, bl0, Wr0, Wl1, bl1, Wr1, Wl2, bl2, Wr2):
    src = adj_t[0].astype(jnp.int32)
    dst = adj_t[1].astype(jnp.int32)
    cnt = jax.ops.segment_sum(jnp.ones((N_EDGES,), jnp.float32), dst,
                              num_segments=N_NODES)
    invc = (1.0 / jnp.clip(cnt, 1.0, None))[:, None]

    h = x
    outs = []
    for (wl, bl, wr) in ((Wl0, bl0, Wr0), (Wl1, bl1, Wr1), (Wl2, bl2, Wr2)):
        agg = jax.ops.segment_sum(jnp.take(h, src, axis=0), dst,
                                  num_segments=N_NODES)
        o = _sage_mm(agg, h, wl, bl[None, :], wr, invc, h.shape[1])
        outs.append(o)
        h = jax.nn.relu(o)
    return jnp.stack(outs, axis=1)


# speedup vs baseline: 6.1111x; 6.1111x over previous
import functools

import jax
import jax.numpy as jnp
from jax import lax
from jax.experimental import pallas as pl
from jax.experimental.pallas import tpu as pltpu
from jax.experimental.pallas import tpu_sc as plsc

N_NODES = 10000
N_PAD = 10016
N_EDGES = 160000
HID = 512
F = 128
EB = 128
NB = 80
E_TILE = NB * EB
ROWS_SC = 624
M_TILE = 1000

_MESH = plsc.VectorSubcoreMesh(core_axis_name="c", subcore_axis_name="s")


def _zero_acc(acc, zbuf, s):
    base = s * ROWS_SC
    for b in range(ROWS_SC // 32):
        pltpu.sync_copy(zbuf, acc.at[pl.ds(base + b * 32, 32)])

    @pl.when(s < 15)
    def _():
        pltpu.sync_copy(zbuf.at[pl.ds(0, 16)], acc.at[pl.ds(base + 608, 16)])

    @pl.when(s == 15)
    def _():
        pltpu.sync_copy(zbuf.at[pl.ds(0, 16)], acc.at[pl.ds(base + 608, 16)])
        pltpu.sync_copy(zbuf, acc.at[pl.ds(base + 624, 32)])


def _writeback(acc, out_at, s):
    @pl.when(s < 15)
    def _():
        pltpu.sync_copy(acc.at[pl.ds(s * ROWS_SC, ROWS_SC)],
                        out_at.at[pl.ds(s * ROWS_SC, ROWS_SC)])

    @pl.when(s == 15)
    def _():
        pltpu.sync_copy(acc.at[pl.ds(15 * ROWS_SC, 640)],
                        out_at.at[pl.ds(15 * ROWS_SC, 640)])


def _make_agg(nchunk):
    nj = nchunk // 2

    def body(hflat, src4, dst3, zeros32, out, acc, sidx0, sidx1, didx,
             rows0, rows1, gsem0, gsem1, isem0, isem1, zbuf):
        c = lax.axis_index("c")
        s = lax.axis_index("s")
        pltpu.sync_copy(zeros32, zbuf)
        pltpu.sync_copy(dst3.at[s], didx)
        sidx = (sidx0, sidx1)
        rows = (rows0, rows1)
        gsems = (gsem0, gsem1)
        isems = (isem0, isem1)
        for jc in range(nj):
            cj = c + 2 * jc
            _zero_acc(acc, zbuf, s)
            plsc.subcore_barrier()
            idescs = [None, None]
            gdescs = [None, None]
            idescs[0] = pltpu.async_copy(src4.at[cj, s, pl.ds(0, 1)], sidx0, isem0)
            idescs[0].wait()
            gdescs[0] = pltpu.async_copy(hflat.at[sidx0.at[0]], rows0, gsem0)
            if NB > 1:
                idescs[1] = pltpu.async_copy(src4.at[cj, s, pl.ds(1, 1)], sidx1, isem1)
            for j in range(NB):
                k = j % 2
                gdescs[k].wait()
                if j + 2 < NB:
                    idescs[k] = pltpu.async_copy(
                        src4.at[cj, s, pl.ds(j + 2, 1)], sidx[k], isems[k])
                if j + 1 < NB:
                    k2 = (j + 1) % 2
                    idescs[k2].wait()
                    gdescs[k2] = pltpu.async_copy(
                        hflat.at[sidx[k2].at[0]], rows[k2], gsems[k2])
                pltpu.sync_copy(rows[k], acc.at[didx.at[j]], add=True)
            plsc.subcore_barrier()
            _writeback(acc, out.at[cj], s)
            if jc + 1 < nj:
                plsc.subcore_barrier()

    return pl.kernel(
        body,
        out_type=jax.ShapeDtypeStruct((nchunk, N_NODES, F), jnp.float32),
        mesh=_MESH,
        scratch_types=[
            pltpu.VMEM_SHARED((N_PAD, F), jnp.float32),
            pltpu.VMEM((1, EB), jnp.int32),
            pltpu.VMEM((1, EB), jnp.int32),
            pltpu.VMEM((NB, EB), jnp.int32),
            pltpu.VMEM((EB, F), jnp.float32),
            pltpu.VMEM((EB, F), jnp.float32),
            pltpu.SemaphoreType.DMA,
            pltpu.SemaphoreType.DMA,
            pltpu.SemaphoreType.DMA,
            pltpu.SemaphoreType.DMA,
            pltpu.VMEM((32, F), jnp.float32),
        ],
    )


def _cnt_body(dstc, ones, zeros32, out, acc, didx, ob, zb):
    c = lax.axis_index("c")
    s = lax.axis_index("s")
    pltpu.sync_copy(ones, ob)
    pltpu.sync_copy(zeros32, zb)
    pltpu.sync_copy(dstc.at[c, s], didx)
    _zero_acc(acc, zb, s)
    plsc.subcore_barrier()
    for j in range(NB // 2):
        pltpu.sync_copy(ob, acc.at[didx.at[j]], add=True)
    plsc.subcore_barrier()
    _writeback(acc, out.at[c], s)


_sc_counts = pl.kernel(
    _cnt_body,
    out_type=jax.ShapeDtypeStruct((2, N_NODES, F), jnp.float32),
    mesh=_MESH,
    scratch_types=[
        pltpu.VMEM_SHARED((N_PAD, F), jnp.float32),
        pltpu.VMEM((NB // 2, EB), jnp.int32),
        pltpu.VMEM((EB, F), jnp.float32),
        pltpu.VMEM((32, F), jnp.float32),
    ],
)

_sc_agg = {2: _make_agg(2), 4: _make_agg(4)}


def _mm_body_relu(agg_ref, h_ref, wl_ref, bl_ref, wr_ref, invc_ref,
                  o1_ref, o2_ref):
    nc = agg_ref.shape[0]
    wl = wl_ref[...]
    wr = wr_ref[...]
    acc = jnp.zeros((M_TILE, HID), jnp.float32)
    for j in range(nc):
        acc += jnp.dot(agg_ref[j], wl[j * F:(j + 1) * F],
                       preferred_element_type=jnp.float32)
    acc *= invc_ref[...]
    for j in range(nc):
        acc += jnp.dot(h_ref[j], wr[j * F:(j + 1) * F],
                       preferred_element_type=jnp.float32)
    acc += bl_ref[...]
    o1_ref[...] = acc
    o2_ref[...] = jnp.transpose(
        jnp.maximum(acc, 0.0).reshape(M_TILE, HID // F, F), (1, 0, 2))


def _mm_body_last(agg_ref, h_ref, wl_ref, bl_ref, wr_ref, invc_ref, o1_ref):
    nc = agg_ref.shape[0]
    wl = wl_ref[...]
    wr = wr_ref[...]
    acc = jnp.zeros((M_TILE, HID), jnp.float32)
    for j in range(nc):
        acc += jnp.dot(agg_ref[j], wl[j * F:(j + 1) * F],
                       preferred_element_type=jnp.float32)
    acc *= invc_ref[...]
    for j in range(nc):
        acc += jnp.dot(h_ref[j], wr[j * F:(j + 1) * F],
                       preferred_element_type=jnp.float32)
    o1_ref[...] = acc + bl_ref[...]


@functools.partial(jax.jit, static_argnames=("nc", "relu"))
def _sage_mm(agg3, h3, wl, bl, wr, invc, nc, relu):
    c_in = nc * F
    grid = (N_NODES // M_TILE,)
    in_specs = [
        pl.BlockSpec((nc, M_TILE, F), lambda i: (0, i, 0)),
        pl.BlockSpec((nc, M_TILE, F), lambda i: (0, i, 0)),
        pl.BlockSpec((c_in, HID), lambda i: (0, 0)),
        pl.BlockSpec((1, HID), lambda i: (0, 0)),
        pl.BlockSpec((c_in, HID), lambda i: (0, 0)),
        pl.BlockSpec((M_TILE, 1), lambda i: (i, 0)),
    ]
    if relu:
        return pl.pallas_call(
            _mm_body_relu,
            grid=grid,
            in_specs=in_specs,
            out_specs=[
                pl.BlockSpec((M_TILE, HID), lambda i: (i, 0)),
                pl.BlockSpec((HID // F, M_TILE, F), lambda i: (0, i, 0)),
            ],
            out_shape=[
                jax.ShapeDtypeStruct((N_NODES, HID), jnp.float32),
                jax.ShapeDtypeStruct((HID // F, N_NODES, F), jnp.float32),
            ],
        )(agg3, h3, wl, bl, wr, invc)
    return pl.pallas_call(
        _mm_body_last,
        grid=grid,
        in_specs=in_specs,
        out_specs=pl.BlockSpec((M_TILE, HID), lambda i: (i, 0)),
        out_shape=jax.ShapeDtypeStruct((N_NODES, HID), jnp.float32),
    )(agg3, h3, wl, bl, wr, invc)


def kernel(x, adj_t, Wl0, bl0, Wr0, Wl1, bl1, Wr1, Wl2, bl2, Wr2):
    src = adj_t[0].astype(jnp.int32)
    dst = adj_t[1].astype(jnp.int32)

    per_tile = N_EDGES // 16
    pad_n = E_TILE - per_tile
    tiles = jnp.arange(16, dtype=jnp.int32)[:, None]
    padi = jnp.arange(pad_n, dtype=jnp.int32)[None, :]
    pad_src = (padi * 67 + tiles * 131) % N_NODES
    pad_dst = N_NODES + (padi + tiles) % 16
    src3 = jnp.concatenate([src.reshape(16, per_tile), pad_src], axis=1)
    dst3 = jnp.concatenate([dst.reshape(16, per_tile), pad_dst], axis=1)
    src3 = src3.reshape(16, NB, EB)
    dst3 = dst3.reshape(16, NB, EB)
    off2 = (jnp.arange(2, dtype=jnp.int32) * N_NODES)[:, None, None, None]
    off4 = (jnp.arange(4, dtype=jnp.int32) * N_NODES)[:, None, None, None]
    src4_2 = src3[None] + off2
    src4_4 = src3[None] + off4

    ones = jnp.ones((EB, F), jnp.float32)
    zeros32 = jnp.zeros((32, F), jnp.float32)

    dstc = jnp.transpose(dst3.reshape(16, 2, NB // 2, EB), (1, 0, 2, 3))
    cnt2 = _sc_counts(dstc, ones, zeros32)
    cnt = cnt2[0, :, 0] + cnt2[1, :, 0]
    invc = (1.0 / jnp.clip(cnt, 1.0, None))[:, None]

    x3 = jnp.transpose(x.reshape(N_NODES, 2, F), (1, 0, 2))
    xflat = x3.reshape(2 * N_NODES, F)

    agg0 = _sc_agg[2](xflat, src4_2, dst3, zeros32)
    o0, h3 = _sage_mm(agg0, x3, Wl0, bl0[None, :], Wr0, invc, 2, True)

    agg1 = _sc_agg[4](h3.reshape(4 * N_NODES, F), src4_4, dst3, zeros32)
    o1, h3 = _sage_mm(agg1, h3, Wl1, bl1[None, :], Wr1, invc, 4, True)

    agg2 = _sc_agg[4](h3.reshape(4 * N_NODES, F), src4_4, dst3, zeros32)
    o2 = _sage_mm(agg2, h3, Wl2, bl2[None, :], Wr2, invc, 4, False)

    return jnp.stack([o0, o1, o2], axis=1)
